# baseline (speedup 1.0000x reference)
import jax
import jax.numpy as jnp
from jax import lax
from jax.experimental import pallas as pl
from jax.experimental.pallas import tpu as pltpu
from jax.experimental.pallas import tpu_sc as plsc

_B = 16384
_V = 1000000
_D = 64
_NC = 2
_NS = 16
_NW = _NC * _NS
_BW = _B // _NW
_CH = 128
_NCH = _BW // _CH
_L = 16


def _mf_body(uid, iid, uembT, iembT, ubias, ibias, gbias, out,
             uidx, iidx, uT, iT, ub, ib, gb, outv, sem):
    c = lax.axis_index("c")
    s = lax.axis_index("s")
    base = (s * _NC + c) * _BW

    pltpu.sync_copy(uid.at[pl.ds(base, _BW)], uidx)
    pltpu.sync_copy(iid.at[pl.ds(base, _BW)], iidx)
    pltpu.sync_copy(gbias, gb)

    for k in range(_NCH):
        sl = pl.ds(k * _CH, _CH)
        pltpu.async_copy(ubias.at[uidx.at[sl]], ub.at[sl], sem)
        pltpu.async_copy(ibias.at[iidx.at[sl]], ib.at[sl], sem)

    def fire(d, carry):
        for k in range(_NCH):
            sl = pl.ds(k * _CH, _CH)
            dsl = pl.ds(pl.multiple_of(d * _BW, _BW) + k * _CH, _CH)
            pltpu.async_copy(uembT.at[d].at[uidx.at[sl]], uT.at[dsl], sem)
            pltpu.async_copy(iembT.at[d].at[iidx.at[sl]], iT.at[dsl], sem)
        return carry

    lax.fori_loop(0, _D, fire, 0)

    pltpu.make_async_copy(uembT.at[0].at[pl.ds(0, _D * _BW)], uT, sem).wait()
    pltpu.make_async_copy(iembT.at[0].at[pl.ds(0, _D * _BW)], iT, sem).wait()
    pltpu.make_async_copy(ubias.at[pl.ds(0, _BW)], ub, sem).wait()
    pltpu.make_async_copy(ibias.at[pl.ds(0, _BW)], ib, sem).wait()

    gvec = gb[...]

    def group(g, carry):
        osl = pl.ds(g * _L, _L)
        accs = [None, None, None, None]
        for d in range(_D):
            csl = pl.ds(d * _BW + g * _L, _L)
            p = uT[csl] * iT[csl]
            accs[d % 4] = p if accs[d % 4] is None else accs[d % 4] + p
        acc = (accs[0] + accs[1]) + (accs[2] + accs[3])
        outv[osl] = acc + (ub[osl] + ib[osl]) + gvec
        return carry

    lax.fori_loop(0, _BW // _L, group, 0)
    pltpu.sync_copy(outv, out.at[pl.ds(base, _BW)])


def kernel(user_ids, item_ids, user_emb, item_emb, user_bias, item_bias, global_bias):
    uid = user_ids.astype(jnp.int32)
    iid = item_ids.astype(jnp.int32)
    uembT = user_emb.T
    iembT = item_emb.T
    ubias = user_bias.reshape(-1)
    ibias = item_bias.reshape(-1)
    gb16 = jnp.broadcast_to(global_bias.astype(jnp.float32), (_L,))
    mesh = plsc.VectorSubcoreMesh(core_axis_name="c", subcore_axis_name="s")
    f = pl.kernel(
        _mf_body,
        mesh=mesh,
        compiler_params=pltpu.CompilerParams(
            needs_layout_passes=False, use_tc_tiling_on_sc=False),
        out_type=jax.ShapeDtypeStruct((_B,), jnp.float32),
        scratch_types=[
            pltpu.VMEM((_BW,), jnp.int32),
            pltpu.VMEM((_BW,), jnp.int32),
            pltpu.VMEM((_D * _BW,), jnp.float32),
            pltpu.VMEM((_D * _BW,), jnp.float32),
            pltpu.VMEM((_BW,), jnp.float32),
            pltpu.VMEM((_BW,), jnp.float32),
            pltpu.VMEM((_L,), jnp.float32),
            pltpu.VMEM((_BW,), jnp.float32),
            pltpu.SemaphoreType.DMA,
        ],
    )
    return f(uid, iid, uembT, iembT, ubias, ibias, gb16)
